# baseline (speedup 1.0000x reference)
import functools

import jax
import jax.numpy as jnp
from jax import lax
from jax.experimental import pallas as pl
from jax.experimental.pallas import tpu as pltpu
from jax.experimental.pallas import tpu_sc as plsc

_SC_PARAMS = pltpu.CompilerParams(needs_layout_passes=False)
_SC_PARAMS_LINEAR = pltpu.CompilerParams(
    needs_layout_passes=False, use_tc_tiling_on_sc=False
)

NC = 2
NS = 16
NW = NC * NS
LANES = 16
CH = 128
NB = 20
RING = 6


def _zeros16():
    return jnp.zeros((LANES,), jnp.float32)


def _node_pad(n):
    return ((n + NW * 8 - 1) // (NW * 8)) * (NW * 8)


def _sc_degrees(ei4, npad):
    nbt = ei4.shape[1]
    nbatch = -(-nbt // NW)
    rng = npad // NS
    mesh = plsc.VectorSubcoreMesh(core_axis_name="c", subcore_axis_name="s")

    @functools.partial(
        pl.kernel,
        out_type=jax.ShapeDtypeStruct((NC, 2, npad), jnp.float32),
        mesh=mesh,
        compiler_params=_SC_PARAMS_LINEAR,
        scratch_types=[
            pltpu.VMEM((2, NB, CH), jnp.int32),
            pltpu.VMEM((2, NB, CH), jnp.int32),
            pltpu.SemaphoreType.DMA,
            pltpu.SemaphoreType.DMA,
            pltpu.VMEM((npad,), jnp.float32),
            pltpu.VMEM((npad,), jnp.float32),
            pltpu.VMEM((rng,), jnp.float32),
            pltpu.VMEM((rng,), jnp.float32),
            pltpu.VMEM((2, rng), jnp.float32),
            pltpu.VMEM((2, rng), jnp.float32),
            pltpu.VMEM_SHARED((NS, 2, npad), jnp.float32),
            pltpu.SemaphoreType.DMA,
            pltpu.SemaphoreType.DMA,
        ],
    )
    def deg_kernel(ei_hbm, out_hbm, idxva, idxvb, isem0, isem1, hist_s, hist_d,
                   accs, accd, tmp0, tmp1, sh, tsem0, tsem1):
        core = lax.axis_index("c")
        sid = lax.axis_index("s")
        wid = sid * NC + core
        z16 = _zeros16()
        ones16 = jnp.full((LANES,), 1.0, jnp.float32)
        idxvs = [idxva, idxvb]
        isems = [isem0, isem1]

        def ifetch(b):
            pltpu.async_copy(ei_hbm.at[0, wid + b * NW], idxvs[b % 2].at[0], isems[b % 2])
            pltpu.async_copy(ei_hbm.at[1, wid + b * NW], idxvs[b % 2].at[1], isems[b % 2])

        def iwait(b):
            pltpu.make_async_copy(ei_hbm.at[0, wid], idxvs[b % 2].at[0], isems[b % 2]).wait()
            pltpu.make_async_copy(ei_hbm.at[1, wid], idxvs[b % 2].at[1], isems[b % 2]).wait()

        ifetch(0)

        @pl.loop(0, npad, step=LANES)
        def _(i):
            hist_s[pl.ds(i, LANES)] = z16
            hist_d[pl.ds(i, LANES)] = z16

        for b in range(nbatch):
            g = wid + b * NW
            @pl.when(g < nbt)
            def _(b=b):
                if b + 1 < nbatch:
                    @pl.when(wid + (b + 1) * NW < nbt)
                    def _():
                        ifetch(b + 1)
                iwait(b)
                idxv = idxvs[b % 2]

                @pl.loop(0, NB)
                def _(k):
                    @pl.loop(0, CH, step=LANES)
                    def _(l):
                        sv = idxv[0, k, pl.ds(l, LANES)]
                        plsc.addupdate_scatter(hist_s, [sv], ones16)
                        dv = idxv[1, k, pl.ds(l, LANES)]
                        plsc.addupdate_scatter(hist_d, [dv], ones16)

        pltpu.sync_copy(hist_s, sh.at[sid, 0])
        pltpu.sync_copy(hist_d, sh.at[sid, 1])
        plsc.subcore_barrier()

        base = sid * rng

        @pl.loop(0, rng, step=LANES)
        def _(i):
            accs[pl.ds(i, LANES)] = z16
            accd[pl.ds(i, LANES)] = z16

        tmps = [tmp0, tmp1]
        tsems = [tsem0, tsem1]

        def _fetch(j):
            par = j % 2
            pltpu.async_copy(sh.at[j, 0, pl.ds(base, rng)], tmps[par].at[0], tsems[par])
            pltpu.async_copy(sh.at[j, 1, pl.ds(base, rng)], tmps[par].at[1], tsems[par])

        def _wait(j):
            par = j % 2
            pltpu.make_async_copy(sh.at[j, 0, pl.ds(base, rng)], tmps[par].at[0], tsems[par]).wait()
            pltpu.make_async_copy(sh.at[j, 1, pl.ds(base, rng)], tmps[par].at[1], tsems[par]).wait()

        _fetch(0)
        for j in range(NS):
            if j + 1 < NS:
                _fetch(j + 1)
            _wait(j)
            par = j % 2

            @pl.loop(0, rng, step=LANES)
            def _(i):
                accs[pl.ds(i, LANES)] = accs[pl.ds(i, LANES)] + tmps[par][0, pl.ds(i, LANES)]
                accd[pl.ds(i, LANES)] = accd[pl.ds(i, LANES)] + tmps[par][1, pl.ds(i, LANES)]

        pltpu.sync_copy(accs, out_hbm.at[core, 0, pl.ds(base, rng)])
        pltpu.sync_copy(accd, out_hbm.at[core, 1, pl.ds(base, rng)])

    return deg_kernel(ei4)


def _sc_aggregate(h, ei4):
    npad, d = h.shape
    nbt = ei4.shape[1]
    nbatch = -(-nbt // NW)
    rows_per_tile = npad // NS
    ring = 2 if d >= 128 else RING
    mesh = plsc.VectorSubcoreMesh(core_axis_name="c", subcore_axis_name="s")

    ahead = ring - 1

    @functools.partial(
        pl.kernel,
        out_type=jax.ShapeDtypeStruct((NC, npad, d), jnp.float32),
        mesh=mesh,
        compiler_params=_SC_PARAMS_LINEAR,
        scratch_types=[
            pltpu.VMEM((2, NB, CH), jnp.int32),
            pltpu.VMEM((2, NB, CH), jnp.int32),
            *[pltpu.VMEM((CH, d), jnp.float32) for _ in range(ring)],
            pltpu.VMEM_SHARED((npad, d), jnp.float32),
            *[pltpu.SemaphoreType.DMA for _ in range(2 * ring + 2)],
        ],
    )
    def agg_kernel(h_hbm, ei_hbm, out_hbm, idxv0, idxv1, *rest):
        bufs = list(rest[:ring])
        acc_sh = rest[ring]
        gsems = list(rest[ring + 1 : ring + 1 + ring])
        ssems = list(rest[ring + 1 + ring : ring + 1 + 2 * ring])
        isems = list(rest[ring + 1 + 2 * ring :])
        idxvs = [idxv0, idxv1]
        core = lax.axis_index("c")
        sid = lax.axis_index("s")
        wid = sid * NC + core
        z16 = _zeros16()

        iobj0a = pltpu.async_copy(ei_hbm.at[0, wid], idxv0.at[0], isems[0])
        iobj0b = pltpu.async_copy(ei_hbm.at[1, wid], idxv0.at[1], isems[0])

        @pl.loop(0, CH)
        def _(r):
            @pl.loop(0, d, step=LANES)
            def _(c):
                bufs[0][r, pl.ds(c, LANES)] = z16

        zbase = sid * rows_per_tile

        @pl.loop(0, rows_per_tile // CH)
        def _(k):
            pltpu.sync_copy(bufs[0], acc_sh.at[pl.ds(zbase + k * CH, CH)])

        plsc.subcore_barrier()
        iobj0a.wait()
        iobj0b.wait()

        def process(b):
            idxv = idxvs[b % 2]
            nxt = b + 1
            has_next = nxt < nbatch
            if has_next:
                pv = idxvs[nxt % 2]
                psem = isems[nxt % 2]
                next_real = wid + nxt * NW < nbt

                @pl.when(next_real)
                def _():
                    pltpu.async_copy(ei_hbm.at[0, wid + nxt * NW], pv.at[0], psem)
                    pltpu.async_copy(ei_hbm.at[1, wid + nxt * NW], pv.at[1], psem)

            gobj = [None] * ring
            sobj = [None] * ring

            def scatter(kk):
                jj = kk % ring
                gobj[jj].wait()
                sobj[jj] = pltpu.async_copy(
                    bufs[jj], acc_sh.at[idxv.at[1, kk]], ssems[jj], add=True
                )

            for k in range(NB):
                j = k % ring
                if k >= ring:
                    sobj[j].wait()
                gobj[j] = pltpu.async_copy(h_hbm.at[idxv.at[0, k]], bufs[j], gsems[j])
                if k >= ahead:
                    scatter(k - ahead)
            for kk in range(NB - ahead, NB):
                scatter(kk)
            for j in range(ring):
                sobj[j].wait()

            if has_next:
                @pl.when(next_real)
                def _():
                    pltpu.make_async_copy(ei_hbm.at[0, wid], pv.at[0], psem).wait()
                    pltpu.make_async_copy(ei_hbm.at[1, wid], pv.at[1], psem).wait()

        process(0)
        for b in range(1, nbatch):
            @pl.when(wid + b * NW < nbt)
            def _(b=b):
                process(b)

        plsc.subcore_barrier()
        pltpu.sync_copy(
            acc_sh.at[pl.ds(zbase, rows_per_tile)],
            out_hbm.at[core, pl.ds(zbase, rows_per_tile)],
        )

    return agg_kernel(h, ei4)


_BR = 2000


def _norms(degt_ref):
    degt = degt_ref[...]
    deg_out = degt[:, 0:1] + degt[:, 2:3]
    deg_in = degt[:, 1:2] + degt[:, 3:4]
    no = jnp.where(deg_out > 0, lax.rsqrt(deg_out), 0.0)
    ni = jnp.where(deg_in > 0, lax.rsqrt(deg_in), 0.0)
    return no, ni


def _hist_spec():
    return pl.BlockSpec((_BR, 4), lambda i: (i, 0))


def _tc1_body(feat_ref, w_ref, hist_ref, out_ref):
    h = jnp.dot(feat_ref[...], w_ref[...], preferred_element_type=jnp.float32)
    out_ref[...] = h * _norms(hist_ref)[0]


def _tc_layer1(feat, w1, hist_t, npad):
    n, f = feat.shape
    h = w1.shape[1]
    grid = n // _BR
    return pl.pallas_call(
        _tc1_body,
        grid=(grid,),
        in_specs=[
            pl.BlockSpec((_BR, f), lambda i: (i, 0)),
            pl.BlockSpec((f, h), lambda i: (0, 0)),
            _hist_spec(),
        ],
        out_specs=pl.BlockSpec((_BR, h), lambda i: (i, 0)),
        out_shape=jax.ShapeDtypeStruct((npad, h), jnp.float32),
    )(feat, w1, hist_t)


def _tc2_body(agg_ref, hist_ref, b1_ref, w2_ref, out_ref):
    no, ni = _norms(hist_ref)
    a = agg_ref[0] + agg_ref[1]
    h1 = a * ni + b1_ref[...]
    r = jnp.maximum(h1, 0.0)
    h2 = jnp.dot(r, w2_ref[...], preferred_element_type=jnp.float32)
    out_ref[...] = h2 * no


def _tc_layer2(agg1, hist_t, b1, w2, n, npad):
    h = agg1.shape[2]
    k = w2.shape[1]
    grid = n // _BR
    return pl.pallas_call(
        _tc2_body,
        grid=(grid,),
        in_specs=[
            pl.BlockSpec((NC, _BR, h), lambda i: (0, i, 0)),
            _hist_spec(),
            pl.BlockSpec((1, h), lambda i: (0, 0)),
            pl.BlockSpec((h, k), lambda i: (0, 0)),
        ],
        out_specs=pl.BlockSpec((_BR, k), lambda i: (i, 0)),
        out_shape=jax.ShapeDtypeStruct((npad, k), jnp.float32),
    )(agg1, hist_t, b1, w2)


def _tc3_body(agg_ref, hist_ref, b2_ref, out_ref):
    a = agg_ref[0] + agg_ref[1]
    out_ref[...] = a * _norms(hist_ref)[1] + b2_ref[...]


def _tc_final(agg2, hist_t, b2, n):
    k = agg2.shape[2]
    grid = n // _BR
    return pl.pallas_call(
        _tc3_body,
        grid=(grid,),
        in_specs=[
            pl.BlockSpec((NC, _BR, k), lambda i: (0, i, 0)),
            _hist_spec(),
            pl.BlockSpec((1, k), lambda i: (0, 0)),
        ],
        out_specs=pl.BlockSpec((_BR, k), lambda i: (i, 0)),
        out_shape=jax.ShapeDtypeStruct((n, k), jnp.float32),
    )(agg2, hist_t, b2)


def kernel(feat, edge_index, W1, b1, W2, b2):
    n, f = feat.shape
    e = edge_index.shape[1]
    npad = _node_pad(n)

    nbt = e // (NB * CH)
    ei4 = edge_index.reshape(2, nbt, NB, CH)

    deg = _sc_degrees(ei4, npad)
    degt = deg.transpose(2, 0, 1).reshape(npad, 4)

    h1 = _tc_layer1(feat, W1, degt, npad)
    agg1 = _sc_aggregate(h1, ei4)
    h2 = _tc_layer2(agg1, degt, b1.reshape(1, -1), W2, n, npad)
    agg2 = _sc_aggregate(h2, ei4)
    return _tc_final(agg2, degt, b2.reshape(1, -1), n)


# speedup vs baseline: 1.0159x; 1.0159x over previous
import functools

import jax
import jax.numpy as jnp
from jax import lax
from jax.experimental import pallas as pl
from jax.experimental.pallas import tpu as pltpu
from jax.experimental.pallas import tpu_sc as plsc

_SC_PARAMS = pltpu.CompilerParams(needs_layout_passes=False)
_SC_PARAMS_LINEAR = pltpu.CompilerParams(
    needs_layout_passes=False, use_tc_tiling_on_sc=False
)

NC = 2
NS = 16
NW = NC * NS
LANES = 16
CH = 128
NB = 20
RING = 6


def _zeros16():
    return jnp.zeros((LANES,), jnp.float32)


def _node_pad(n):
    return ((n + NW * 8 - 1) // (NW * 8)) * (NW * 8)


def _sc_degrees(ei4, npad):
    nbt = ei4.shape[1]
    nbatch = -(-nbt // NW)
    rng = npad // NS
    mesh = plsc.VectorSubcoreMesh(core_axis_name="c", subcore_axis_name="s")

    @functools.partial(
        pl.kernel,
        out_type=jax.ShapeDtypeStruct((NC, 2, npad), jnp.float32),
        mesh=mesh,
        compiler_params=_SC_PARAMS_LINEAR,
        scratch_types=[
            pltpu.VMEM((2, NB, CH), jnp.int32),
            pltpu.VMEM((2, NB, CH), jnp.int32),
            pltpu.SemaphoreType.DMA,
            pltpu.SemaphoreType.DMA,
            pltpu.VMEM((npad,), jnp.float32),
            pltpu.VMEM((npad,), jnp.float32),
            pltpu.VMEM((rng,), jnp.float32),
            pltpu.VMEM((rng,), jnp.float32),
            pltpu.VMEM((2, rng), jnp.float32),
            pltpu.VMEM((2, rng), jnp.float32),
            pltpu.VMEM_SHARED((NS, 2, npad), jnp.float32),
            pltpu.SemaphoreType.DMA,
            pltpu.SemaphoreType.DMA,
        ],
    )
    def deg_kernel(ei_hbm, out_hbm, idxva, idxvb, isem0, isem1, hist_s, hist_d,
                   accs, accd, tmp0, tmp1, sh, tsem0, tsem1):
        core = lax.axis_index("c")
        sid = lax.axis_index("s")
        wid = sid * NC + core
        z16 = _zeros16()
        ones16 = jnp.full((LANES,), 1.0, jnp.float32)
        idxv = idxva

        @pl.loop(0, npad, step=LANES)
        def _(i):
            hist_s[pl.ds(i, LANES)] = z16
            hist_d[pl.ds(i, LANES)] = z16

        for b in range(nbatch):
            g = wid + b * NW
            @pl.when(g < nbt)
            def _():
                pltpu.sync_copy(ei_hbm.at[0, g], idxv.at[0])
                pltpu.sync_copy(ei_hbm.at[1, g], idxv.at[1])

                @pl.loop(0, NB)
                def _(k):
                    @pl.loop(0, CH, step=LANES)
                    def _(l):
                        sv = idxv[0, k, pl.ds(l, LANES)]
                        plsc.addupdate_scatter(hist_s, [sv], ones16)
                        dv = idxv[1, k, pl.ds(l, LANES)]
                        plsc.addupdate_scatter(hist_d, [dv], ones16)

        pltpu.sync_copy(hist_s, sh.at[sid, 0])
        pltpu.sync_copy(hist_d, sh.at[sid, 1])
        plsc.subcore_barrier()

        base = sid * rng

        @pl.loop(0, rng, step=LANES)
        def _(i):
            accs[pl.ds(i, LANES)] = z16
            accd[pl.ds(i, LANES)] = z16

        tmps = [tmp0, tmp1]
        tsems = [tsem0, tsem1]

        def _fetch(j):
            par = j % 2
            pltpu.async_copy(sh.at[j, 0, pl.ds(base, rng)], tmps[par].at[0], tsems[par])
            pltpu.async_copy(sh.at[j, 1, pl.ds(base, rng)], tmps[par].at[1], tsems[par])

        def _wait(j):
            par = j % 2
            pltpu.make_async_copy(sh.at[j, 0, pl.ds(base, rng)], tmps[par].at[0], tsems[par]).wait()
            pltpu.make_async_copy(sh.at[j, 1, pl.ds(base, rng)], tmps[par].at[1], tsems[par]).wait()

        _fetch(0)
        for j in range(NS):
            if j + 1 < NS:
                _fetch(j + 1)
            _wait(j)
            par = j % 2

            @pl.loop(0, rng, step=LANES)
            def _(i):
                accs[pl.ds(i, LANES)] = accs[pl.ds(i, LANES)] + tmps[par][0, pl.ds(i, LANES)]
                accd[pl.ds(i, LANES)] = accd[pl.ds(i, LANES)] + tmps[par][1, pl.ds(i, LANES)]

        pltpu.sync_copy(accs, out_hbm.at[core, 0, pl.ds(base, rng)])
        pltpu.sync_copy(accd, out_hbm.at[core, 1, pl.ds(base, rng)])

    return deg_kernel(ei4)


def _sc_aggregate(h, ei4):
    npad, d = h.shape
    nbt = ei4.shape[1]
    nbatch = -(-nbt // NW)
    rows_per_tile = npad // NS
    ring = 2 if d >= 128 else RING
    mesh = plsc.VectorSubcoreMesh(core_axis_name="c", subcore_axis_name="s")

    ahead = ring - 1

    @functools.partial(
        pl.kernel,
        out_type=jax.ShapeDtypeStruct((NC, npad, d), jnp.float32),
        mesh=mesh,
        compiler_params=_SC_PARAMS_LINEAR,
        scratch_types=[
            pltpu.VMEM((2, NB, CH), jnp.int32),
            pltpu.VMEM((2, NB, CH), jnp.int32),
            *[pltpu.VMEM((CH, d), jnp.float32) for _ in range(ring)],
            pltpu.VMEM_SHARED((npad, d), jnp.float32),
            *[pltpu.SemaphoreType.DMA for _ in range(2 * ring + 2)],
        ],
    )
    def agg_kernel(h_hbm, ei_hbm, out_hbm, idxv0, idxv1, *rest):
        bufs = list(rest[:ring])
        acc_sh = rest[ring]
        gsems = list(rest[ring + 1 : ring + 1 + ring])
        ssems = list(rest[ring + 1 + ring : ring + 1 + 2 * ring])
        isems = list(rest[ring + 1 + 2 * ring :])
        idxvs = [idxv0, idxv1]
        core = lax.axis_index("c")
        sid = lax.axis_index("s")
        wid = sid * NC + core
        z16 = _zeros16()

        iobj0a = pltpu.async_copy(ei_hbm.at[0, wid], idxv0.at[0], isems[0])
        iobj0b = pltpu.async_copy(ei_hbm.at[1, wid], idxv0.at[1], isems[0])

        @pl.loop(0, CH)
        def _(r):
            @pl.loop(0, d, step=LANES)
            def _(c):
                bufs[0][r, pl.ds(c, LANES)] = z16

        zbase = sid * rows_per_tile

        @pl.loop(0, rows_per_tile // CH)
        def _(k):
            pltpu.sync_copy(bufs[0], acc_sh.at[pl.ds(zbase + k * CH, CH)])

        plsc.subcore_barrier()
        iobj0a.wait()
        iobj0b.wait()

        def process(b):
            idxv = idxvs[b % 2]
            nxt = b + 1
            has_next = nxt < nbatch
            if has_next:
                pv = idxvs[nxt % 2]
                psem = isems[nxt % 2]
                next_real = wid + nxt * NW < nbt

                @pl.when(next_real)
                def _():
                    pltpu.async_copy(ei_hbm.at[0, wid + nxt * NW], pv.at[0], psem)
                    pltpu.async_copy(ei_hbm.at[1, wid + nxt * NW], pv.at[1], psem)

            gobj = [None] * ring
            sobj = [None] * ring

            def scatter(kk):
                jj = kk % ring
                gobj[jj].wait()
                sobj[jj] = pltpu.async_copy(
                    bufs[jj], acc_sh.at[idxv.at[1, kk]], ssems[jj], add=True
                )

            for k in range(NB):
                j = k % ring
                if k >= ring:
                    sobj[j].wait()
                gobj[j] = pltpu.async_copy(h_hbm.at[idxv.at[0, k]], bufs[j], gsems[j])
                if k >= ahead:
                    scatter(k - ahead)
            for kk in range(NB - ahead, NB):
                scatter(kk)
            for j in range(ring):
                sobj[j].wait()

            if has_next:
                @pl.when(next_real)
                def _():
                    pltpu.make_async_copy(ei_hbm.at[0, wid], pv.at[0], psem).wait()
                    pltpu.make_async_copy(ei_hbm.at[1, wid], pv.at[1], psem).wait()

        process(0)
        for b in range(1, nbatch):
            @pl.when(wid + b * NW < nbt)
            def _(b=b):
                process(b)

        plsc.subcore_barrier()
        pltpu.sync_copy(
            acc_sh.at[pl.ds(zbase, rows_per_tile)],
            out_hbm.at[core, pl.ds(zbase, rows_per_tile)],
        )

    return agg_kernel(h, ei4)


_BR = 2000


def _norms(degt_ref):
    degt = degt_ref[...]
    deg_out = degt[:, 0:1] + degt[:, 2:3]
    deg_in = degt[:, 1:2] + degt[:, 3:4]
    no = jnp.where(deg_out > 0, lax.rsqrt(deg_out), 0.0)
    ni = jnp.where(deg_in > 0, lax.rsqrt(deg_in), 0.0)
    return no, ni


def _hist_spec():
    return pl.BlockSpec((_BR, 4), lambda i: (i, 0))


def _tc1_body(feat_ref, w_ref, hist_ref, out_ref):
    h = jnp.dot(feat_ref[...], w_ref[...], preferred_element_type=jnp.float32)
    out_ref[...] = h * _norms(hist_ref)[0]


def _tc_layer1(feat, w1, hist_t, npad):
    n, f = feat.shape
    h = w1.shape[1]
    grid = n // _BR
    return pl.pallas_call(
        _tc1_body,
        grid=(grid,),
        in_specs=[
            pl.BlockSpec((_BR, f), lambda i: (i, 0)),
            pl.BlockSpec((f, h), lambda i: (0, 0)),
            _hist_spec(),
        ],
        out_specs=pl.BlockSpec((_BR, h), lambda i: (i, 0)),
        out_shape=jax.ShapeDtypeStruct((npad, h), jnp.float32),
    )(feat, w1, hist_t)


def _tc2_body(agg_ref, hist_ref, b1_ref, w2_ref, out_ref):
    no, ni = _norms(hist_ref)
    a = agg_ref[0] + agg_ref[1]
    h1 = a * ni + b1_ref[...]
    r = jnp.maximum(h1, 0.0)
    h2 = jnp.dot(r, w2_ref[...], preferred_element_type=jnp.float32)
    out_ref[...] = h2 * no


def _tc_layer2(agg1, hist_t, b1, w2, n, npad):
    h = agg1.shape[2]
    k = w2.shape[1]
    grid = n // _BR
    return pl.pallas_call(
        _tc2_body,
        grid=(grid,),
        in_specs=[
            pl.BlockSpec((NC, _BR, h), lambda i: (0, i, 0)),
            _hist_spec(),
            pl.BlockSpec((1, h), lambda i: (0, 0)),
            pl.BlockSpec((h, k), lambda i: (0, 0)),
        ],
        out_specs=pl.BlockSpec((_BR, k), lambda i: (i, 0)),
        out_shape=jax.ShapeDtypeStruct((npad, k), jnp.float32),
    )(agg1, hist_t, b1, w2)


def _tc3_body(agg_ref, hist_ref, b2_ref, out_ref):
    a = agg_ref[0] + agg_ref[1]
    out_ref[...] = a * _norms(hist_ref)[1] + b2_ref[...]


def _tc_final(agg2, hist_t, b2, n):
    k = agg2.shape[2]
    grid = n // _BR
    return pl.pallas_call(
        _tc3_body,
        grid=(grid,),
        in_specs=[
            pl.BlockSpec((NC, _BR, k), lambda i: (0, i, 0)),
            _hist_spec(),
            pl.BlockSpec((1, k), lambda i: (0, 0)),
        ],
        out_specs=pl.BlockSpec((_BR, k), lambda i: (i, 0)),
        out_shape=jax.ShapeDtypeStruct((n, k), jnp.float32),
    )(agg2, hist_t, b2)


def kernel(feat, edge_index, W1, b1, W2, b2):
    n, f = feat.shape
    e = edge_index.shape[1]
    npad = _node_pad(n)

    nbt = e // (NB * CH)
    ei4 = edge_index.reshape(2, nbt, NB, CH)

    deg = _sc_degrees(ei4, npad)
    degt = deg.transpose(2, 0, 1).reshape(npad, 4)

    h1 = _tc_layer1(feat, W1, degt, npad)
    agg1 = _sc_aggregate(h1, ei4)
    h2 = _tc_layer2(agg1, degt, b1.reshape(1, -1), W2, n, npad)
    agg2 = _sc_aggregate(h2, ei4)
    return _tc_final(agg2, degt, b2.reshape(1, -1), n)
